# baseline (speedup 1.0000x reference)
import jax
import jax.numpy as jnp
from jax import lax
from jax.experimental import pallas as pl
from jax.experimental.pallas import tpu as pltpu
from jax.experimental.pallas import tpu_sc as plsc

E = 64
K = 2
D_IN = 768
D_OUT = 768
T = 2048
P = T * K
TILE = 128
N_TILES = P // TILE
NW = 96
SC_CORES = 2
SC_SUBCORES = 16
SC_WORKERS = SC_CORES * SC_SUBCORES


def _gating_kernel(x_ref, wg_ref, pos0_ref, pos1_ref, g0_ref, g1_ref,
                   wt_ref, we_ref, wlo_ref, whi_ref, wfirst_ref, loss_ref):
    logits = jnp.dot(x_ref[...], wg_ref[...],
                     preferred_element_type=jnp.float32)
    eiota = lax.broadcasted_iota(jnp.int32, (T, E), 1)
    l1 = jnp.max(logits, axis=1, keepdims=True)
    i1 = jnp.min(jnp.where(logits == l1, eiota, E), axis=1, keepdims=True)
    is1 = eiota == i1
    l2 = jnp.max(jnp.where(is1, -jnp.inf, logits), axis=1, keepdims=True)
    i2 = jnp.min(jnp.where((logits == l2) & (~is1), eiota, E),
                 axis=1, keepdims=True)
    is2 = eiota == i2
    sexp = jnp.sum(jnp.exp(logits - l1), axis=1, keepdims=True)
    p1 = 1.0 / sexp
    p2 = jnp.exp(l2 - l1) / sexp
    den = p1 + p2 + 1e-6
    g0 = p1 / den
    g1 = p2 / den
    oh0 = is1.astype(jnp.float32)
    oh1 = is2.astype(jnp.float32)
    oh = oh0 + oh1
    tri = (lax.broadcasted_iota(jnp.int32, (T, T), 0) >
           lax.broadcasted_iota(jnp.int32, (T, T), 1)).astype(jnp.bfloat16)
    cb = jnp.dot(tri, oh.astype(jnp.bfloat16),
                 preferred_element_type=jnp.float32)
    r0 = jnp.sum(oh0 * cb, axis=1, keepdims=True)
    r1 = jnp.sum(oh1 * cb, axis=1, keepdims=True)
    counts = jnp.sum(oh, axis=0, keepdims=True)
    imp = jnp.sum(oh0 * g0 + oh1 * g1, axis=0, keepdims=True)
    load = jnp.sum(oh0 * (g0 > 0.0).astype(jnp.float32) +
                   oh1 * (g1 > 0.0).astype(jnp.float32), axis=0, keepdims=True)
    up = (lax.broadcasted_iota(jnp.int32, (E, E), 0) <
          lax.broadcasted_iota(jnp.int32, (E, E), 1)).astype(jnp.float32)
    off = jnp.dot(counts, up, preferred_element_type=jnp.float32,
                  precision=lax.Precision.HIGHEST)
    off_inc = off + counts
    off0 = jnp.sum(jnp.where(is1, off, 0.0), axis=1, keepdims=True)
    off1 = jnp.sum(jnp.where(is2, off, 0.0), axis=1, keepdims=True)
    pos0 = (off0 + r0).astype(jnp.int32)
    pos1 = (off1 + r1).astype(jnp.int32)
    pos0_ref[...] = pos0
    pos1_ref[...] = pos1
    g0_ref[...] = g0
    g1_ref[...] = g1

    inv_tile = 1.0 / TILE
    tile_lo = jnp.floor(off * inv_tile)
    tile_hi = jnp.floor((off_inc - 1.0) * inv_tile)
    n_e = jnp.where(counts > 0.0, tile_hi - tile_lo + 1.0, 0.0)
    cum = jnp.dot(n_e, up, preferred_element_type=jnp.float32,
                  precision=lax.Precision.HIGHEST)
    total = jnp.sum(n_e, axis=1, keepdims=True)
    w_io = lax.broadcasted_iota(jnp.int32, (NW, E), 0).astype(jnp.float32)
    e_io = lax.broadcasted_iota(jnp.int32, (NW, E), 1).astype(jnp.float32)
    w_col = w_io[:, 0:1]
    e_w = jnp.sum((cum <= w_io).astype(jnp.float32), axis=1,
                  keepdims=True) - 1.0
    e_last = jnp.sum((cum <= total - 1.0).astype(jnp.float32),
                     axis=1, keepdims=True) - 1.0
    valid = w_col < total
    e_w = jnp.where(valid, e_w, e_last)
    oh_e = (e_io == e_w).astype(jnp.float32)
    sel = lambda row: jnp.sum(oh_e * row, axis=1, keepdims=True)
    off_w = sel(off)
    offp1_w = sel(off_inc)
    cum_w = sel(cum)
    tile_w = sel(tile_lo) + (w_col - cum_w)
    oh_el = ((lax.broadcasted_iota(jnp.int32, (1, E), 1)
              ).astype(jnp.float32) == e_last).astype(jnp.float32)
    tile_last = (jnp.sum(oh_el * tile_lo, axis=1, keepdims=True) +
                 (total - 1.0 - jnp.sum(oh_el * cum, axis=1, keepdims=True)))
    tile_w = jnp.where(valid, jnp.clip(tile_w, 0.0, N_TILES - 1.0), tile_last)
    lo_w = jnp.where(valid, jnp.maximum(off_w, tile_w * TILE), 0.0)
    hi_w = jnp.where(valid, jnp.minimum(offp1_w, tile_w * TILE + TILE), 0.0)
    first_w = valid & (off_w <= tile_w * TILE)
    wt_ref[...] = tile_w.astype(jnp.int32)
    we_ref[...] = e_w.astype(jnp.int32)
    wlo_ref[...] = lo_w.astype(jnp.int32)
    whi_ref[...] = hi_w.astype(jnp.int32)
    wfirst_ref[...] = first_w.astype(jnp.int32)

    def cv2(v):
        m = jnp.sum(v, axis=1, keepdims=True) / E
        var = jnp.sum((v - m) ** 2, axis=1, keepdims=True) / (E - 1)
        return var / (m * m + 1e-10)

    loss_ref[...] = (cv2(imp) + cv2(load)) * 1e-2


def _run_gating(x, w_gate):
    return pl.pallas_call(
        _gating_kernel,
        out_shape=[
            jax.ShapeDtypeStruct((T, 1), jnp.int32),
            jax.ShapeDtypeStruct((T, 1), jnp.int32),
            jax.ShapeDtypeStruct((T, 1), jnp.float32),
            jax.ShapeDtypeStruct((T, 1), jnp.float32),
            jax.ShapeDtypeStruct((NW, 1), jnp.int32),
            jax.ShapeDtypeStruct((NW, 1), jnp.int32),
            jax.ShapeDtypeStruct((NW, 1), jnp.int32),
            jax.ShapeDtypeStruct((NW, 1), jnp.int32),
            jax.ShapeDtypeStruct((NW, 1), jnp.int32),
            jax.ShapeDtypeStruct((1, 1), jnp.float32),
        ],
    )(x, w_gate)


def _dispatch_body(x_hbm, pos0_hbm, pos1_hbm, xs_hbm,
                   tok_v, pos_v, rows_v, sem_g, sem_s):
    wid = lax.axis_index("s") * SC_CORES + lax.axis_index("c")
    tpw = T // SC_WORKERS
    base_t = wid * tpw
    pltpu.sync_copy(pos0_hbm.at[pl.ds(base_t, tpw)], pos_v.at[pl.ds(0, tpw)])
    pltpu.sync_copy(pos1_hbm.at[pl.ds(base_t, tpw)], pos_v.at[pl.ds(tpw, tpw)])
    ii = lax.iota(jnp.int32, 16)
    for cth in range(tpw // 16):
        tok = base_t + cth * 16 + ii
        tok_v[pl.ds(cth * 16, 16)] = tok
        tok_v[pl.ds(tpw + cth * 16, 16)] = tok
    pltpu.async_copy(x_hbm.at[tok_v], rows_v, sem_g).wait()
    pltpu.async_copy(rows_v, xs_hbm.at[pos_v], sem_s).wait()


def _run_dispatch(x, pos0, pos1):
    mesh = plsc.VectorSubcoreMesh(core_axis_name="c", subcore_axis_name="s",
                                  num_cores=SC_CORES, num_subcores=SC_SUBCORES)
    npw = P // SC_WORKERS
    f = pl.kernel(
        _dispatch_body,
        out_type=jax.ShapeDtypeStruct((P, D_IN), jnp.float32),
        mesh=mesh,
        scratch_types=[
            pltpu.VMEM((npw,), jnp.int32),
            pltpu.VMEM((npw,), jnp.int32),
            pltpu.VMEM((npw, D_IN), jnp.float32),
            pltpu.SemaphoreType.DMA,
            pltpu.SemaphoreType.DMA,
        ],
    )
    return f(x, pos0, pos1)


def _gate_scatter_kernel(pos0_ref, pos1_ref, g0_ref, g1_ref, gs_ref):
    pos0 = pos0_ref[...]
    pos1 = pos1_ref[...]
    g0 = g0_ref[...]
    g1 = g1_ref[...]
    for sc in range(P // 512):
        siota = sc * 512 + lax.broadcasted_iota(jnp.int32, (T, 512), 1)
        gsc = jnp.sum(jnp.where(pos0 == siota, g0, 0.0) +
                      jnp.where(pos1 == siota, g1, 0.0),
                      axis=0, keepdims=True)
        gs_ref[:, pl.ds(sc * 512, 512)] = gsc


def _run_gate_scatter(pos0, pos1, g0, g1):
    return pl.pallas_call(
        _gate_scatter_kernel,
        out_shape=jax.ShapeDtypeStruct((1, P), jnp.float32),
    )(pos0, pos1, g0, g1)


def _expert_mm_kernel(tile_ref, expert_ref, lo_ref, hi_ref, first_ref,
                      xs_ref, w1_ref, w2_ref, b_ref, gs_ref, out_ref):
    w = pl.program_id(0)
    tile = tile_ref[w]
    lo = lo_ref[w]
    hi = hi_ref[w]

    @pl.when(hi > lo)
    def _compute():
        rel_lo = lo - tile * TILE
        rel_hi = hi - tile * TILE
        rio = lax.broadcasted_iota(jnp.int32, (TILE, 1), 0)
        active = (rio >= rel_lo) & (rio < rel_hi)
        sl = pl.ds(tile * TILE, TILE)
        xm = jnp.where(active, xs_ref[sl, :], 0.0)
        grow = gs_ref[0:1, pl.ds(tile * TILE, TILE)]
        eye = (lax.broadcasted_iota(jnp.int32, (TILE, TILE), 0) ==
               lax.broadcasted_iota(jnp.int32, (TILE, TILE), 1))
        g = jnp.sum(jnp.where(eye, grow, 0.0), axis=1, keepdims=True)
        xb = xm.astype(jnp.bfloat16)
        z = g * (jnp.dot(xb[:, :D_IN // 2], w1_ref[0].astype(jnp.bfloat16),
                         preferred_element_type=jnp.float32) +
                 jnp.dot(xb[:, D_IN // 2:], w2_ref[0].astype(jnp.bfloat16),
                         preferred_element_type=jnp.float32))
        z = z + jnp.where(active, g * b_ref[0], 0.0)
        first = first_ref[w] > 0
        out_ref[sl, :] = jnp.where(first, z, out_ref[sl, :] + z)


def _run_expert_mm(xs, expert_w, expert_b, gs,
                   wi_tile, wi_expert, wi_lo, wi_hi, wi_first):
    grid_spec = pltpu.PrefetchScalarGridSpec(
        num_scalar_prefetch=5,
        grid=(NW,),
        in_specs=[
            pl.BlockSpec((P, D_IN), lambda w, t, e, lo, hi, f: (0, 0)),
            pl.BlockSpec((1, D_IN // 2, D_OUT),
                         lambda w, t, e, lo, hi, f: (e[w], 0, 0)),
            pl.BlockSpec((1, D_IN // 2, D_OUT),
                         lambda w, t, e, lo, hi, f: (e[w], 1, 0)),
            pl.BlockSpec((1, 1, D_OUT),
                         lambda w, t, e, lo, hi, f: (e[w], 0, 0)),
            pl.BlockSpec((1, P), lambda w, t, e, lo, hi, f: (0, 0)),
        ],
        out_specs=pl.BlockSpec((P, D_OUT),
                               lambda w, t, e, lo, hi, f: (0, 0)),
    )
    return pl.pallas_call(
        _expert_mm_kernel,
        grid_spec=grid_spec,
        out_shape=jax.ShapeDtypeStruct((P, D_OUT), jnp.float32),
    )(wi_tile, wi_expert, wi_lo, wi_hi, wi_first, xs, expert_w, expert_w,
      expert_b.reshape(E, 1, D_OUT), gs)


def _combine_body(pout_hbm, pos0_hbm, pos1_hbm, y_hbm,
                  idx0a_v, idx1a_v, idx0b_v, idx1b_v,
                  r0a_v, r1a_v, r0b_v, r1b_v, y_v,
                  sem0a, sem1a, sem0b, sem1b):
    wid = lax.axis_index("s") * SC_CORES + lax.axis_index("c")
    tpw = T // SC_WORKERS
    base = wid * tpw
    pltpu.sync_copy(pos0_hbm.at[pl.ds(base, 32)], idx0a_v)
    pltpu.sync_copy(pos1_hbm.at[pl.ds(base, 32)], idx1a_v)
    pltpu.sync_copy(pos0_hbm.at[pl.ds(base + 32, 32)], idx0b_v)
    pltpu.sync_copy(pos1_hbm.at[pl.ds(base + 32, 32)], idx1b_v)
    cp0a = pltpu.async_copy(pout_hbm.at[idx0a_v], r0a_v, sem0a)
    cp1a = pltpu.async_copy(pout_hbm.at[idx1a_v], r1a_v, sem1a)
    cp0b = pltpu.async_copy(pout_hbm.at[idx0b_v], r0b_v, sem0b)
    cp1b = pltpu.async_copy(pout_hbm.at[idx1b_v], r1b_v, sem1b)

    def add_store(r0_v, r1_v, out_base):
        def body(i, carry):
            for c in range(D_OUT // 16):
                sl = pl.ds(c * 16, 16)
                y_v[i, sl] = r0_v[i, sl] + r1_v[i, sl]
            return carry

        lax.fori_loop(0, 32, body, 0)
        pltpu.sync_copy(y_v, y_hbm.at[pl.ds(out_base, 32)])

    cp0a.wait()
    cp1a.wait()
    add_store(r0a_v, r1a_v, base)
    cp0b.wait()
    cp1b.wait()
    add_store(r0b_v, r1b_v, base + 32)


def _run_combine(pout, pos0, pos1):
    mesh = plsc.VectorSubcoreMesh(core_axis_name="c", subcore_axis_name="s",
                                  num_cores=SC_CORES, num_subcores=SC_SUBCORES)
    f = pl.kernel(
        _combine_body,
        out_type=jax.ShapeDtypeStruct((T, D_OUT), jnp.float32),
        mesh=mesh,
        scratch_types=[
            pltpu.VMEM((32,), jnp.int32),
            pltpu.VMEM((32,), jnp.int32),
            pltpu.VMEM((32,), jnp.int32),
            pltpu.VMEM((32,), jnp.int32),
            pltpu.VMEM((32, D_OUT), jnp.float32),
            pltpu.VMEM((32, D_OUT), jnp.float32),
            pltpu.VMEM((32, D_OUT), jnp.float32),
            pltpu.VMEM((32, D_OUT), jnp.float32),
            pltpu.VMEM((32, D_OUT), jnp.float32),
            pltpu.SemaphoreType.DMA,
            pltpu.SemaphoreType.DMA,
            pltpu.SemaphoreType.DMA,
            pltpu.SemaphoreType.DMA,
        ],
    )
    return f(pout, pos0, pos1)


def kernel(x, w_gate, expert_w, expert_b):
    (pos0, pos1, g0, g1,
     wt, we, wlo, whi, wfirst, loss) = _run_gating(x, w_gate)
    pos0f = pos0.reshape(T)
    pos1f = pos1.reshape(T)
    xs = _run_dispatch(x, pos0f, pos1f)
    gs = _run_gate_scatter(pos0, pos1, g0, g1)
    pout = _run_expert_mm(xs, expert_w, expert_b, gs,
                          wt.reshape(NW), we.reshape(NW), wlo.reshape(NW),
                          whi.reshape(NW), wfirst.reshape(NW))
    y = _run_combine(pout, pos0f, pos1f)
    return y, loss[0, 0]


# speedup vs baseline: 1.0071x; 1.0071x over previous
import jax
import jax.numpy as jnp
from jax import lax
from jax.experimental import pallas as pl
from jax.experimental.pallas import tpu as pltpu
from jax.experimental.pallas import tpu_sc as plsc

E = 64
K = 2
D_IN = 768
D_OUT = 768
T = 2048
P = T * K
TILE = 128
N_TILES = P // TILE
NW = 96
SC_CORES = 2
SC_SUBCORES = 16
SC_WORKERS = SC_CORES * SC_SUBCORES


def _gating_kernel(x_ref, wg_ref, pos0_ref, pos1_ref, g0_ref, g1_ref,
                   wt_ref, we_ref, wlo_ref, whi_ref, wfirst_ref, loss_ref):
    logits = jnp.dot(x_ref[...], wg_ref[...],
                     preferred_element_type=jnp.float32)
    eiota = lax.broadcasted_iota(jnp.int32, (T, E), 1)
    l1 = jnp.max(logits, axis=1, keepdims=True)
    i1 = jnp.min(jnp.where(logits == l1, eiota, E), axis=1, keepdims=True)
    is1 = eiota == i1
    l2 = jnp.max(jnp.where(is1, -jnp.inf, logits), axis=1, keepdims=True)
    i2 = jnp.min(jnp.where((logits == l2) & (~is1), eiota, E),
                 axis=1, keepdims=True)
    is2 = eiota == i2
    sexp = jnp.sum(jnp.exp(logits - l1), axis=1, keepdims=True)
    p1 = 1.0 / sexp
    p2 = jnp.exp(l2 - l1) / sexp
    den = p1 + p2 + 1e-6
    g0 = p1 / den
    g1 = p2 / den
    oh0 = is1.astype(jnp.float32)
    oh1 = is2.astype(jnp.float32)
    oh = oh0 + oh1
    tri = (lax.broadcasted_iota(jnp.int32, (T, T), 0) >
           lax.broadcasted_iota(jnp.int32, (T, T), 1)).astype(jnp.bfloat16)
    cb = jnp.dot(tri, oh.astype(jnp.bfloat16),
                 preferred_element_type=jnp.float32)
    r0 = jnp.sum(oh0 * cb, axis=1, keepdims=True)
    r1 = jnp.sum(oh1 * cb, axis=1, keepdims=True)
    counts = jnp.sum(oh, axis=0, keepdims=True)
    imp = jnp.sum(oh0 * g0 + oh1 * g1, axis=0, keepdims=True)
    load = jnp.sum(oh0 * (g0 > 0.0).astype(jnp.float32) +
                   oh1 * (g1 > 0.0).astype(jnp.float32), axis=0, keepdims=True)
    up = (lax.broadcasted_iota(jnp.int32, (E, E), 0) <
          lax.broadcasted_iota(jnp.int32, (E, E), 1)).astype(jnp.float32)
    off = jnp.dot(counts, up, preferred_element_type=jnp.float32,
                  precision=lax.Precision.HIGHEST)
    off_inc = off + counts
    off0 = jnp.sum(jnp.where(is1, off, 0.0), axis=1, keepdims=True)
    off1 = jnp.sum(jnp.where(is2, off, 0.0), axis=1, keepdims=True)
    pos0 = (off0 + r0).astype(jnp.int32)
    pos1 = (off1 + r1).astype(jnp.int32)
    pos0_ref[...] = pos0
    pos1_ref[...] = pos1
    g0_ref[...] = g0
    g1_ref[...] = g1

    inv_tile = 1.0 / TILE
    tile_lo = jnp.floor(off * inv_tile)
    tile_hi = jnp.floor((off_inc - 1.0) * inv_tile)
    n_e = jnp.where(counts > 0.0, tile_hi - tile_lo + 1.0, 0.0)
    cum = jnp.dot(n_e, up, preferred_element_type=jnp.float32,
                  precision=lax.Precision.HIGHEST)
    total = jnp.sum(n_e, axis=1, keepdims=True)
    w_io = lax.broadcasted_iota(jnp.int32, (NW, E), 0).astype(jnp.float32)
    e_io = lax.broadcasted_iota(jnp.int32, (NW, E), 1).astype(jnp.float32)
    w_col = w_io[:, 0:1]
    e_w = jnp.sum((cum <= w_io).astype(jnp.float32), axis=1,
                  keepdims=True) - 1.0
    e_last = jnp.sum((cum <= total - 1.0).astype(jnp.float32),
                     axis=1, keepdims=True) - 1.0
    valid = w_col < total
    e_w = jnp.where(valid, e_w, e_last)
    oh_e = (e_io == e_w).astype(jnp.float32)
    sel = lambda row: jnp.sum(oh_e * row, axis=1, keepdims=True)
    off_w = sel(off)
    offp1_w = sel(off_inc)
    cum_w = sel(cum)
    tile_w = sel(tile_lo) + (w_col - cum_w)
    oh_el = ((lax.broadcasted_iota(jnp.int32, (1, E), 1)
              ).astype(jnp.float32) == e_last).astype(jnp.float32)
    tile_last = (jnp.sum(oh_el * tile_lo, axis=1, keepdims=True) +
                 (total - 1.0 - jnp.sum(oh_el * cum, axis=1, keepdims=True)))
    tile_w = jnp.where(valid, jnp.clip(tile_w, 0.0, N_TILES - 1.0), tile_last)
    lo_w = jnp.where(valid, jnp.maximum(off_w, tile_w * TILE), 0.0)
    hi_w = jnp.where(valid, jnp.minimum(offp1_w, tile_w * TILE + TILE), 0.0)
    first_w = valid & (off_w <= tile_w * TILE)
    wt_ref[...] = tile_w.astype(jnp.int32)
    we_ref[...] = e_w.astype(jnp.int32)
    wlo_ref[...] = lo_w.astype(jnp.int32)
    whi_ref[...] = hi_w.astype(jnp.int32)
    wfirst_ref[...] = first_w.astype(jnp.int32)

    def cv2(v):
        m = jnp.sum(v, axis=1, keepdims=True) / E
        var = jnp.sum((v - m) ** 2, axis=1, keepdims=True) / (E - 1)
        return var / (m * m + 1e-10)

    loss_ref[...] = (cv2(imp) + cv2(load)) * 1e-2


def _run_gating(x, w_gate):
    return pl.pallas_call(
        _gating_kernel,
        out_shape=[
            jax.ShapeDtypeStruct((T, 1), jnp.int32),
            jax.ShapeDtypeStruct((T, 1), jnp.int32),
            jax.ShapeDtypeStruct((T, 1), jnp.float32),
            jax.ShapeDtypeStruct((T, 1), jnp.float32),
            jax.ShapeDtypeStruct((NW, 1), jnp.int32),
            jax.ShapeDtypeStruct((NW, 1), jnp.int32),
            jax.ShapeDtypeStruct((NW, 1), jnp.int32),
            jax.ShapeDtypeStruct((NW, 1), jnp.int32),
            jax.ShapeDtypeStruct((NW, 1), jnp.int32),
            jax.ShapeDtypeStruct((1, 1), jnp.float32),
        ],
    )(x, w_gate)


def _dispatch_body(x_hbm, pos0_hbm, pos1_hbm, xs_hbm,
                   tok_v, pos_v, rows_v, sem_g, sem_s):
    wid = lax.axis_index("s") * SC_CORES + lax.axis_index("c")
    tpw = T // SC_WORKERS
    base_t = wid * tpw
    pltpu.sync_copy(pos0_hbm.at[pl.ds(base_t, tpw)], pos_v.at[pl.ds(0, tpw)])
    pltpu.sync_copy(pos1_hbm.at[pl.ds(base_t, tpw)], pos_v.at[pl.ds(tpw, tpw)])
    ii = lax.iota(jnp.int32, 16)
    for cth in range(tpw // 16):
        tok = base_t + cth * 16 + ii
        tok_v[pl.ds(cth * 16, 16)] = tok
        tok_v[pl.ds(tpw + cth * 16, 16)] = tok
    pltpu.async_copy(x_hbm.at[tok_v], rows_v, sem_g).wait()
    pltpu.async_copy(rows_v, xs_hbm.at[pos_v], sem_s).wait()


def _run_dispatch(x, pos0, pos1):
    mesh = plsc.VectorSubcoreMesh(core_axis_name="c", subcore_axis_name="s",
                                  num_cores=SC_CORES, num_subcores=SC_SUBCORES)
    npw = P // SC_WORKERS
    f = pl.kernel(
        _dispatch_body,
        out_type=jax.ShapeDtypeStruct((P, D_IN), jnp.float32),
        mesh=mesh,
        scratch_types=[
            pltpu.VMEM((npw,), jnp.int32),
            pltpu.VMEM((npw,), jnp.int32),
            pltpu.VMEM((npw, D_IN), jnp.float32),
            pltpu.SemaphoreType.DMA,
            pltpu.SemaphoreType.DMA,
        ],
    )
    return f(x, pos0, pos1)


def _gate_scatter_kernel(pos0_ref, pos1_ref, g0_ref, g1_ref, gs_ref):
    pos0 = pos0_ref[...]
    pos1 = pos1_ref[...]
    g0 = g0_ref[...]
    g1 = g1_ref[...]
    for sc in range(P // 512):
        siota = sc * 512 + lax.broadcasted_iota(jnp.int32, (T, 512), 1)
        gsc = jnp.sum(jnp.where(pos0 == siota, g0, 0.0) +
                      jnp.where(pos1 == siota, g1, 0.0),
                      axis=0, keepdims=True)
        gs_ref[:, pl.ds(sc * 512, 512)] = gsc


def _run_gate_scatter(pos0, pos1, g0, g1):
    return pl.pallas_call(
        _gate_scatter_kernel,
        out_shape=jax.ShapeDtypeStruct((1, P), jnp.float32),
    )(pos0, pos1, g0, g1)


def _expert_mm_kernel(tile_ref, expert_ref, lo_ref, hi_ref, first_ref,
                      xs_ref, w_ref, b_ref, gs_ref, out_ref):
    w = pl.program_id(0)
    tile = tile_ref[w]
    lo = lo_ref[w]
    hi = hi_ref[w]

    @pl.when(hi > lo)
    def _compute():
        rel_lo = lo - tile * TILE
        rel_hi = hi - tile * TILE
        rio = lax.broadcasted_iota(jnp.int32, (TILE, 1), 0)
        active = (rio >= rel_lo) & (rio < rel_hi)
        sl = pl.ds(tile * TILE, TILE)
        xm = jnp.where(active, xs_ref[sl, :], 0.0)
        grow = gs_ref[0:1, pl.ds(tile * TILE, TILE)]
        eye = (lax.broadcasted_iota(jnp.int32, (TILE, TILE), 0) ==
               lax.broadcasted_iota(jnp.int32, (TILE, TILE), 1))
        g = jnp.sum(jnp.where(eye, grow, 0.0), axis=1, keepdims=True)
        z = g * jnp.dot(xm.astype(jnp.bfloat16),
                        w_ref[0].astype(jnp.bfloat16),
                        preferred_element_type=jnp.float32)
        z = z + jnp.where(active, g * b_ref[0], 0.0)
        first = first_ref[w] > 0
        out_ref[sl, :] = jnp.where(first, z, out_ref[sl, :] + z)


def _run_expert_mm(xs, expert_w, expert_b, gs,
                   wi_tile, wi_expert, wi_lo, wi_hi, wi_first):
    grid_spec = pltpu.PrefetchScalarGridSpec(
        num_scalar_prefetch=5,
        grid=(NW,),
        in_specs=[
            pl.BlockSpec((P, D_IN), lambda w, t, e, lo, hi, f: (0, 0)),
            pl.BlockSpec((1, D_IN, D_OUT),
                         lambda w, t, e, lo, hi, f: (e[w], 0, 0)),
            pl.BlockSpec((1, 1, D_OUT),
                         lambda w, t, e, lo, hi, f: (e[w], 0, 0)),
            pl.BlockSpec((1, P), lambda w, t, e, lo, hi, f: (0, 0)),
        ],
        out_specs=pl.BlockSpec((P, D_OUT),
                               lambda w, t, e, lo, hi, f: (0, 0)),
    )
    return pl.pallas_call(
        _expert_mm_kernel,
        grid_spec=grid_spec,
        out_shape=jax.ShapeDtypeStruct((P, D_OUT), jnp.float32),
    )(wi_tile, wi_expert, wi_lo, wi_hi, wi_first, xs, expert_w,
      expert_b.reshape(E, 1, D_OUT), gs)


def _combine_body(pout_hbm, pos0_hbm, pos1_hbm, y_hbm,
                  idx0a_v, idx1a_v, idx0b_v, idx1b_v,
                  r0a_v, r1a_v, r0b_v, r1b_v, y_v,
                  sem0a, sem1a, sem0b, sem1b):
    wid = lax.axis_index("s") * SC_CORES + lax.axis_index("c")
    tpw = T // SC_WORKERS
    base = wid * tpw
    pltpu.sync_copy(pos0_hbm.at[pl.ds(base, 32)], idx0a_v)
    pltpu.sync_copy(pos1_hbm.at[pl.ds(base, 32)], idx1a_v)
    pltpu.sync_copy(pos0_hbm.at[pl.ds(base + 32, 32)], idx0b_v)
    pltpu.sync_copy(pos1_hbm.at[pl.ds(base + 32, 32)], idx1b_v)
    cp0a = pltpu.async_copy(pout_hbm.at[idx0a_v], r0a_v, sem0a)
    cp1a = pltpu.async_copy(pout_hbm.at[idx1a_v], r1a_v, sem1a)
    cp0b = pltpu.async_copy(pout_hbm.at[idx0b_v], r0b_v, sem0b)
    cp1b = pltpu.async_copy(pout_hbm.at[idx1b_v], r1b_v, sem1b)

    def add_store(r0_v, r1_v, out_base):
        def body(i, carry):
            for c in range(D_OUT // 16):
                sl = pl.ds(c * 16, 16)
                y_v[i, sl] = r0_v[i, sl] + r1_v[i, sl]
            return carry

        lax.fori_loop(0, 32, body, 0)
        pltpu.sync_copy(y_v, y_hbm.at[pl.ds(out_base, 32)])

    cp0a.wait()
    cp1a.wait()
    add_store(r0a_v, r1a_v, base)
    cp0b.wait()
    cp1b.wait()
    add_store(r0b_v, r1b_v, base + 32)


def _run_combine(pout, pos0, pos1):
    mesh = plsc.VectorSubcoreMesh(core_axis_name="c", subcore_axis_name="s",
                                  num_cores=SC_CORES, num_subcores=SC_SUBCORES)
    f = pl.kernel(
        _combine_body,
        out_type=jax.ShapeDtypeStruct((T, D_OUT), jnp.float32),
        mesh=mesh,
        scratch_types=[
            pltpu.VMEM((32,), jnp.int32),
            pltpu.VMEM((32,), jnp.int32),
            pltpu.VMEM((32,), jnp.int32),
            pltpu.VMEM((32,), jnp.int32),
            pltpu.VMEM((32, D_OUT), jnp.float32),
            pltpu.VMEM((32, D_OUT), jnp.float32),
            pltpu.VMEM((32, D_OUT), jnp.float32),
            pltpu.VMEM((32, D_OUT), jnp.float32),
            pltpu.VMEM((32, D_OUT), jnp.float32),
            pltpu.SemaphoreType.DMA,
            pltpu.SemaphoreType.DMA,
            pltpu.SemaphoreType.DMA,
            pltpu.SemaphoreType.DMA,
        ],
    )
    return f(pout, pos0, pos1)


def kernel(x, w_gate, expert_w, expert_b):
    (pos0, pos1, g0, g1,
     wt, we, wlo, whi, wfirst, loss) = _run_gating(x, w_gate)
    pos0f = pos0.reshape(T)
    pos1f = pos1.reshape(T)
    xs = _run_dispatch(x, pos0f, pos1f)
    gs = _run_gate_scatter(pos0, pos1, g0, g1)
    pout = _run_expert_mm(xs, expert_w, expert_b, gs,
                          wt.reshape(NW), we.reshape(NW), wlo.reshape(NW),
                          whi.reshape(NW), wfirst.reshape(NW))
    y = _run_combine(pout, pos0f, pos1f)
    return y, loss[0, 0]


# speedup vs baseline: 1.0081x; 1.0010x over previous
import jax
import jax.numpy as jnp
from jax import lax
from jax.experimental import pallas as pl
from jax.experimental.pallas import tpu as pltpu
from jax.experimental.pallas import tpu_sc as plsc

E = 64
K = 2
D_IN = 768
D_OUT = 768
T = 2048
P = T * K
TILE = 128
N_TILES = P // TILE
NW = 96
SC_CORES = 2
SC_SUBCORES = 16
SC_WORKERS = SC_CORES * SC_SUBCORES


def _gating_kernel(x_ref, wg_ref, pos0_ref, pos1_ref, g0_ref, g1_ref,
                   wt_ref, we_ref, wlo_ref, whi_ref, wfirst_ref, loss_ref):
    logits = jnp.dot(x_ref[...], wg_ref[...],
                     preferred_element_type=jnp.float32)
    eiota = lax.broadcasted_iota(jnp.int32, (T, E), 1)
    l1 = jnp.max(logits, axis=1, keepdims=True)
    i1 = jnp.min(jnp.where(logits == l1, eiota, E), axis=1, keepdims=True)
    is1 = eiota == i1
    l2 = jnp.max(jnp.where(is1, -jnp.inf, logits), axis=1, keepdims=True)
    i2 = jnp.min(jnp.where((logits == l2) & (~is1), eiota, E),
                 axis=1, keepdims=True)
    is2 = eiota == i2
    sexp = jnp.sum(jnp.exp(logits - l1), axis=1, keepdims=True)
    p1 = 1.0 / sexp
    p2 = jnp.exp(l2 - l1) / sexp
    den = p1 + p2 + 1e-6
    g0 = p1 / den
    g1 = p2 / den
    oh0 = is1.astype(jnp.float32)
    oh1 = is2.astype(jnp.float32)
    oh = oh0 + oh1
    tri = (lax.broadcasted_iota(jnp.int32, (T, T), 0) >
           lax.broadcasted_iota(jnp.int32, (T, T), 1)).astype(jnp.bfloat16)
    cb = jnp.dot(tri, oh.astype(jnp.bfloat16),
                 preferred_element_type=jnp.float32)
    r0 = jnp.sum(oh0 * cb, axis=1, keepdims=True)
    r1 = jnp.sum(oh1 * cb, axis=1, keepdims=True)
    counts = jnp.sum(oh, axis=0, keepdims=True)
    imp = jnp.sum(oh0 * g0 + oh1 * g1, axis=0, keepdims=True)
    load = jnp.sum(oh0 * (g0 > 0.0).astype(jnp.float32) +
                   oh1 * (g1 > 0.0).astype(jnp.float32), axis=0, keepdims=True)
    up = (lax.broadcasted_iota(jnp.int32, (E, E), 0) <
          lax.broadcasted_iota(jnp.int32, (E, E), 1)).astype(jnp.float32)
    off = jnp.dot(counts, up, preferred_element_type=jnp.float32,
                  precision=lax.Precision.HIGHEST)
    off_inc = off + counts
    off0 = jnp.sum(jnp.where(is1, off, 0.0), axis=1, keepdims=True)
    off1 = jnp.sum(jnp.where(is2, off, 0.0), axis=1, keepdims=True)
    pos0 = (off0 + r0).astype(jnp.int32)
    pos1 = (off1 + r1).astype(jnp.int32)
    pos0_ref[...] = pos0
    pos1_ref[...] = pos1
    g0_ref[...] = g0
    g1_ref[...] = g1

    inv_tile = 1.0 / TILE
    tile_lo = jnp.floor(off * inv_tile)
    tile_hi = jnp.floor((off_inc - 1.0) * inv_tile)
    n_e = jnp.where(counts > 0.0, tile_hi - tile_lo + 1.0, 0.0)
    cum = jnp.dot(n_e, up, preferred_element_type=jnp.float32,
                  precision=lax.Precision.HIGHEST)
    total = jnp.sum(n_e, axis=1, keepdims=True)
    w_io = lax.broadcasted_iota(jnp.int32, (NW, E), 0).astype(jnp.float32)
    e_io = lax.broadcasted_iota(jnp.int32, (NW, E), 1).astype(jnp.float32)
    w_col = w_io[:, 0:1]
    e_w = jnp.sum((cum <= w_io).astype(jnp.float32), axis=1,
                  keepdims=True) - 1.0
    e_last = jnp.sum((cum <= total - 1.0).astype(jnp.float32),
                     axis=1, keepdims=True) - 1.0
    valid = w_col < total
    e_w = jnp.where(valid, e_w, e_last)
    oh_e = (e_io == e_w).astype(jnp.float32)
    sel = lambda row: jnp.sum(oh_e * row, axis=1, keepdims=True)
    off_w = sel(off)
    offp1_w = sel(off_inc)
    cum_w = sel(cum)
    tile_w = sel(tile_lo) + (w_col - cum_w)
    oh_el = ((lax.broadcasted_iota(jnp.int32, (1, E), 1)
              ).astype(jnp.float32) == e_last).astype(jnp.float32)
    tile_last = (jnp.sum(oh_el * tile_lo, axis=1, keepdims=True) +
                 (total - 1.0 - jnp.sum(oh_el * cum, axis=1, keepdims=True)))
    tile_w = jnp.where(valid, jnp.clip(tile_w, 0.0, N_TILES - 1.0), tile_last)
    lo_w = jnp.where(valid, jnp.maximum(off_w, tile_w * TILE), 0.0)
    hi_w = jnp.where(valid, jnp.minimum(offp1_w, tile_w * TILE + TILE), 0.0)
    first_w = valid & (off_w <= tile_w * TILE)
    wt_ref[...] = tile_w.astype(jnp.int32)
    we_ref[...] = e_w.astype(jnp.int32)
    wlo_ref[...] = lo_w.astype(jnp.int32)
    whi_ref[...] = hi_w.astype(jnp.int32)
    wfirst_ref[...] = first_w.astype(jnp.int32)

    def cv2(v):
        m = jnp.sum(v, axis=1, keepdims=True) / E
        var = jnp.sum((v - m) ** 2, axis=1, keepdims=True) / (E - 1)
        return var / (m * m + 1e-10)

    loss_ref[...] = (cv2(imp) + cv2(load)) * 1e-2


def _run_gating(x, w_gate):
    return pl.pallas_call(
        _gating_kernel,
        out_shape=[
            jax.ShapeDtypeStruct((T, 1), jnp.int32),
            jax.ShapeDtypeStruct((T, 1), jnp.int32),
            jax.ShapeDtypeStruct((T, 1), jnp.float32),
            jax.ShapeDtypeStruct((T, 1), jnp.float32),
            jax.ShapeDtypeStruct((NW, 1), jnp.int32),
            jax.ShapeDtypeStruct((NW, 1), jnp.int32),
            jax.ShapeDtypeStruct((NW, 1), jnp.int32),
            jax.ShapeDtypeStruct((NW, 1), jnp.int32),
            jax.ShapeDtypeStruct((NW, 1), jnp.int32),
            jax.ShapeDtypeStruct((1, 1), jnp.float32),
        ],
    )(x, w_gate)


def _dispatch_body(x_hbm, pos0_hbm, pos1_hbm, xs_hbm,
                   tok_v, pos_v, rows_v, sem_g, sem_s):
    wid = lax.axis_index("s") * SC_CORES + lax.axis_index("c")
    tpw = T // SC_WORKERS
    base_t = wid * tpw
    pltpu.sync_copy(pos0_hbm.at[pl.ds(base_t, tpw)], pos_v.at[pl.ds(0, tpw)])
    pltpu.sync_copy(pos1_hbm.at[pl.ds(base_t, tpw)], pos_v.at[pl.ds(tpw, tpw)])
    ii = lax.iota(jnp.int32, 16)
    for cth in range(tpw // 16):
        tok = base_t + cth * 16 + ii
        tok_v[pl.ds(cth * 16, 16)] = tok
        tok_v[pl.ds(tpw + cth * 16, 16)] = tok
    pltpu.async_copy(x_hbm.at[tok_v], rows_v, sem_g).wait()
    pltpu.async_copy(rows_v, xs_hbm.at[pos_v], sem_s).wait()


def _run_dispatch(x, pos0, pos1):
    mesh = plsc.VectorSubcoreMesh(core_axis_name="c", subcore_axis_name="s",
                                  num_cores=SC_CORES, num_subcores=SC_SUBCORES)
    npw = P // SC_WORKERS
    f = pl.kernel(
        _dispatch_body,
        out_type=jax.ShapeDtypeStruct((P, D_IN), jnp.float32),
        mesh=mesh,
        scratch_types=[
            pltpu.VMEM((npw,), jnp.int32),
            pltpu.VMEM((npw,), jnp.int32),
            pltpu.VMEM((npw, D_IN), jnp.float32),
            pltpu.SemaphoreType.DMA,
            pltpu.SemaphoreType.DMA,
        ],
    )
    return f(x, pos0, pos1)


def _gate_scatter_kernel(pos0_ref, pos1_ref, g0_ref, g1_ref, gs_ref):
    pos0 = pos0_ref[...]
    pos1 = pos1_ref[...]
    g0 = g0_ref[...]
    g1 = g1_ref[...]
    for sc in range(P // 512):
        siota = sc * 512 + lax.broadcasted_iota(jnp.int32, (T, 512), 1)
        gsc = jnp.sum(jnp.where(pos0 == siota, g0, 0.0) +
                      jnp.where(pos1 == siota, g1, 0.0),
                      axis=0, keepdims=True)
        gs_ref[:, pl.ds(sc * 512, 512)] = gsc


def _run_gate_scatter(pos0, pos1, g0, g1):
    return pl.pallas_call(
        _gate_scatter_kernel,
        out_shape=jax.ShapeDtypeStruct((1, P), jnp.float32),
    )(pos0, pos1, g0, g1)


def _expert_mm_kernel(tile_ref, expert_ref, lo_ref, hi_ref, first_ref,
                      xs_ref, w_ref, b_ref, gs_ref, out_ref):
    w = pl.program_id(0)
    tile = tile_ref[w, 0]
    lo = lo_ref[w, 0]
    hi = hi_ref[w, 0]

    @pl.when(hi > lo)
    def _compute():
        rel_lo = lo - tile * TILE
        rel_hi = hi - tile * TILE
        rio = lax.broadcasted_iota(jnp.int32, (TILE, 1), 0)
        active = (rio >= rel_lo) & (rio < rel_hi)
        sl = pl.ds(tile * TILE, TILE)
        xm = jnp.where(active, xs_ref[sl, :], 0.0)
        grow = gs_ref[0:1, pl.ds(tile * TILE, TILE)]
        eye = (lax.broadcasted_iota(jnp.int32, (TILE, TILE), 0) ==
               lax.broadcasted_iota(jnp.int32, (TILE, TILE), 1))
        g = jnp.sum(jnp.where(eye, grow, 0.0), axis=1, keepdims=True)
        z = g * jnp.dot(xm.astype(jnp.bfloat16),
                        w_ref[0].astype(jnp.bfloat16),
                        preferred_element_type=jnp.float32)
        z = z + jnp.where(active, g * b_ref[0], 0.0)
        first = first_ref[w, 0] > 0
        out_ref[sl, :] = jnp.where(first, z, out_ref[sl, :] + z)


def _run_expert_mm(xs, expert_w, expert_b, gs,
                   wi_tile, wi_expert, wi_lo, wi_hi, wi_first):
    grid_spec = pltpu.PrefetchScalarGridSpec(
        num_scalar_prefetch=5,
        grid=(NW,),
        in_specs=[
            pl.BlockSpec((P, D_IN), lambda w, t, e, lo, hi, f: (0, 0)),
            pl.BlockSpec((1, D_IN, D_OUT),
                         lambda w, t, e, lo, hi, f: (e[w, 0], 0, 0)),
            pl.BlockSpec((1, 1, D_OUT),
                         lambda w, t, e, lo, hi, f: (e[w, 0], 0, 0)),
            pl.BlockSpec((1, P), lambda w, t, e, lo, hi, f: (0, 0)),
        ],
        out_specs=pl.BlockSpec((P, D_OUT),
                               lambda w, t, e, lo, hi, f: (0, 0)),
    )
    return pl.pallas_call(
        _expert_mm_kernel,
        grid_spec=grid_spec,
        out_shape=jax.ShapeDtypeStruct((P, D_OUT), jnp.float32),
    )(wi_tile, wi_expert, wi_lo, wi_hi, wi_first, xs, expert_w,
      expert_b.reshape(E, 1, D_OUT), gs)


def _combine_body(pout_hbm, pos0_hbm, pos1_hbm, y_hbm,
                  idx0a_v, idx1a_v, idx0b_v, idx1b_v,
                  r0a_v, r1a_v, r0b_v, r1b_v, y_v,
                  sem0a, sem1a, sem0b, sem1b):
    wid = lax.axis_index("s") * SC_CORES + lax.axis_index("c")
    tpw = T // SC_WORKERS
    base = wid * tpw
    pltpu.sync_copy(pos0_hbm.at[pl.ds(base, 32)], idx0a_v)
    pltpu.sync_copy(pos1_hbm.at[pl.ds(base, 32)], idx1a_v)
    pltpu.sync_copy(pos0_hbm.at[pl.ds(base + 32, 32)], idx0b_v)
    pltpu.sync_copy(pos1_hbm.at[pl.ds(base + 32, 32)], idx1b_v)
    cp0a = pltpu.async_copy(pout_hbm.at[idx0a_v], r0a_v, sem0a)
    cp1a = pltpu.async_copy(pout_hbm.at[idx1a_v], r1a_v, sem1a)
    cp0b = pltpu.async_copy(pout_hbm.at[idx0b_v], r0b_v, sem0b)
    cp1b = pltpu.async_copy(pout_hbm.at[idx1b_v], r1b_v, sem1b)

    def add_store(r0_v, r1_v, out_base):
        def body(i, carry):
            for c in range(D_OUT // 16):
                sl = pl.ds(c * 16, 16)
                y_v[i, sl] = r0_v[i, sl] + r1_v[i, sl]
            return carry

        lax.fori_loop(0, 32, body, 0)
        pltpu.sync_copy(y_v, y_hbm.at[pl.ds(out_base, 32)])

    cp0a.wait()
    cp1a.wait()
    add_store(r0a_v, r1a_v, base)
    cp0b.wait()
    cp1b.wait()
    add_store(r0b_v, r1b_v, base + 32)


def _run_combine(pout, pos0, pos1):
    mesh = plsc.VectorSubcoreMesh(core_axis_name="c", subcore_axis_name="s",
                                  num_cores=SC_CORES, num_subcores=SC_SUBCORES)
    f = pl.kernel(
        _combine_body,
        out_type=jax.ShapeDtypeStruct((T, D_OUT), jnp.float32),
        mesh=mesh,
        scratch_types=[
            pltpu.VMEM((32,), jnp.int32),
            pltpu.VMEM((32,), jnp.int32),
            pltpu.VMEM((32,), jnp.int32),
            pltpu.VMEM((32,), jnp.int32),
            pltpu.VMEM((32, D_OUT), jnp.float32),
            pltpu.VMEM((32, D_OUT), jnp.float32),
            pltpu.VMEM((32, D_OUT), jnp.float32),
            pltpu.VMEM((32, D_OUT), jnp.float32),
            pltpu.VMEM((32, D_OUT), jnp.float32),
            pltpu.SemaphoreType.DMA,
            pltpu.SemaphoreType.DMA,
            pltpu.SemaphoreType.DMA,
            pltpu.SemaphoreType.DMA,
        ],
    )
    return f(pout, pos0, pos1)


def kernel(x, w_gate, expert_w, expert_b):
    (pos0, pos1, g0, g1,
     wt, we, wlo, whi, wfirst, loss) = _run_gating(x, w_gate)
    pos0f = pos0.reshape(T)
    pos1f = pos1.reshape(T)
    xs = _run_dispatch(x, pos0f, pos1f)
    gs = _run_gate_scatter(pos0, pos1, g0, g1)
    pout = _run_expert_mm(xs, expert_w, expert_b, gs,
                          wt, we, wlo, whi, wfirst)
    y = _run_combine(pout, pos0f, pos1f)
    return y, loss[0, 0]
